# Initial kernel scaffold; baseline (speedup 1.0000x reference)
import jax
import jax.numpy as jnp
from jax.experimental import pallas as pl


def kernel(pos_features, size_features, exist_features, batch, edge_index, pos_W, pos_b, size_W, size_b, exist_table, enc_W, enc_b, node_W, node_b, conv_W0, conv_b0, conv_W1, conv_b1, conv_W2, conv_b2, agg_W, agg_b, mu_W, mu_b, var_W, var_b):
    raise NotImplementedError("write your pallas kernel here")


# speedup vs baseline: 8.5755x; 8.5755x over previous
import functools

import jax
import jax.numpy as jnp
from jax import lax
from jax.experimental import pallas as pl
from jax.experimental.pallas import tpu as pltpu
from jax.experimental.pallas import tpu_sc as plsc

N = 15360
E = 491520
B = 256
FD = 128
HD = 64
LD = 64
NORD = 120

NC = 2
NS = 16
HN = N // NC
AR = HN + 128
ZPS = AR // NS
CPS = HN // NS
EPS = E // NS
KCH = 128
NCHS = EPS // KCH

NEG = float("-inf")


def _sc_scatter_rows(hp, row, col, zeros128):
  mesh = plsc.VectorSubcoreMesh(core_axis_name="c", subcore_axis_name="s")

  @functools.partial(
      pl.kernel,
      mesh=mesh,
      out_type=jax.ShapeDtypeStruct((NC, HN, FD), jnp.float32),
      scratch_types=[
          pltpu.VMEM((KCH,), jnp.int32),
          pltpu.VMEM((KCH,), jnp.int32),
          pltpu.VMEM((KCH, FD), jnp.float32),
          pltpu.VMEM_SHARED((AR, FD), jnp.float32),
          pltpu.SemaphoreType.DMA,
      ],
  )
  def kern(h_hbm, row_hbm, col_hbm, z_hbm, out_hbm, row_v, col_v, rows_v,
           acc, sem):
    c = lax.axis_index("c")
    s = lax.axis_index("s")
    lo = c * HN
    pltpu.sync_copy(z_hbm.at[pl.ds(s * ZPS, ZPS)], acc.at[pl.ds(s * ZPS, ZPS)])
    plsc.subcore_barrier()
    base = s * EPS

    def body(i, carry):
      off = base + i * KCH
      pltpu.sync_copy(row_hbm.at[pl.ds(off, KCH)], row_v)
      pltpu.sync_copy(col_hbm.at[pl.ds(off, KCH)], col_v)
      for j in range(KCH // 16):
        cv = col_v[pl.ds(j * 16, 16)]
        rel = cv - lo
        ok = (rel >= 0) & (rel < HN)
        col_v[pl.ds(j * 16, 16)] = jnp.where(ok, rel, HN)
      pltpu.async_copy(h_hbm.at[row_v], rows_v, sem).wait()
      pltpu.sync_copy(rows_v, acc.at[col_v], add=True)
      return carry

    lax.fori_loop(0, NCHS, body, 0)
    plsc.subcore_barrier()
    pltpu.sync_copy(acc.at[pl.ds(s * CPS, CPS)],
                    out_hbm.at[c, pl.ds(s * CPS, CPS)])

  return kern(hp, row, col, zeros128)


def _sc_degree(col, ones128, zeros128):
  mesh = plsc.VectorSubcoreMesh(core_axis_name="c", subcore_axis_name="s")

  @functools.partial(
      pl.kernel,
      mesh=mesh,
      out_type=jax.ShapeDtypeStruct((NC, HN, FD), jnp.float32),
      scratch_types=[
          pltpu.VMEM((KCH,), jnp.int32),
          pltpu.VMEM((KCH, FD), jnp.float32),
          pltpu.VMEM_SHARED((AR, FD), jnp.float32),
      ],
  )
  def kern(col_hbm, ones_hbm, z_hbm, out_hbm, col_v, ones_v, acc):
    c = lax.axis_index("c")
    s = lax.axis_index("s")
    lo = c * HN
    pltpu.sync_copy(ones_hbm, ones_v)
    pltpu.sync_copy(z_hbm.at[pl.ds(s * ZPS, ZPS)], acc.at[pl.ds(s * ZPS, ZPS)])
    plsc.subcore_barrier()
    base = s * EPS

    def body(i, carry):
      off = base + i * KCH
      pltpu.sync_copy(col_hbm.at[pl.ds(off, KCH)], col_v)
      for j in range(KCH // 16):
        cv = col_v[pl.ds(j * 16, 16)]
        rel = cv - lo
        ok = (rel >= 0) & (rel < HN)
        col_v[pl.ds(j * 16, 16)] = jnp.where(ok, rel, HN)
      pltpu.sync_copy(ones_v, acc.at[col_v], add=True)
      return carry

    lax.fori_loop(0, NCHS, body, 0)
    plsc.subcore_barrier()
    pltpu.sync_copy(acc.at[pl.ds(s * CPS, CPS)],
                    out_hbm.at[c, pl.ds(s * CPS, CPS)])

  return kern(col, ones128, zeros128)


def _shift_down(x, k, fill):
  pad = jnp.full((k,) + x.shape[1:], fill, x.dtype)
  return jnp.concatenate([pad, x[:-k]], axis=0)


def _shift_up(x, k, fill):
  pad = jnp.full((k,) + x.shape[1:], fill, x.dtype)
  return jnp.concatenate([x[k:], pad], axis=0)


def _seg_suffix_max(h, batch_f):
  s = h
  k = 1
  while k < N:
    bsh = _shift_up(batch_f, k, jnp.int32(-7))
    hsh = _shift_up(s, k, NEG)
    s = jnp.where(bsh == batch_f, jnp.maximum(s, hsh), s)
    k *= 2
  return s


def _segmax(h, batch_f, batch_r, start_r):
  iota_b = lax.broadcasted_iota(jnp.int32, (B, 1), 0)
  chunk = N // 8
  parts = []
  for fc in range(0, h.shape[1], 64):
    s = _seg_suffix_max(h[:, fc:fc + 64], batch_f[:, fc:fc + 64])
    acc = jnp.zeros((B, s.shape[1]), jnp.float32)
    for j in range(8):
      sl = slice(j * chunk, (j + 1) * chunk)
      fj = jnp.where(
          (batch_r[:, sl] == iota_b) & (start_r[:, sl] != 0),
          jnp.float32(1.0), jnp.float32(0.0))
      acc = acc + jnp.dot(fj, s[sl], preferred_element_type=jnp.float32)
    parts.append(acc)
  return jnp.concatenate(parts, axis=1)


def _start_row(batch_r):
  pad = jnp.full((1, 1), -7, jnp.int32)
  prev = jnp.concatenate([pad, batch_r[:, :-1]], axis=1)
  return (batch_r != prev).astype(jnp.int32)


def _rshift(x, k, fill):
  pad = jnp.full((1, k), fill, x.dtype)
  return jnp.concatenate([pad, x[:, :-k]], axis=1)


def _order_body(batch_r_ref, order_ref):
  batch_r = batch_r_ref[...]
  iota_r = lax.broadcasted_iota(jnp.int32, (1, N), 1)
  prev = _rshift(batch_r, 1, jnp.int32(-7))
  is_start = batch_r != prev
  f = jnp.where(is_start, iota_r, jnp.int32(-1))
  k = 1
  while k < N:
    f = jnp.maximum(f, _rshift(f, k, jnp.int32(-1)))
    k *= 2
  order_ref[...] = iota_r - f


def _ex_body(ex_ref, tab_ref, order_ref, encw_ref, encb_ref, ex2_ref):
  relu = lambda v: jnp.maximum(v, jnp.float32(0.0))
  tab = tab_ref[...]
  exi = ex_ref[...]
  ex = relu(jnp.where(exi == 1, tab[1:2, :], tab[0:1, :]))
  oh = jnp.where(
      order_ref[...] == lax.broadcasted_iota(jnp.int32, (1, NORD), 1),
      jnp.float32(1.0), jnp.float32(0.0))
  encw = encw_ref[...]
  ex2_ref[...] = relu(
      jnp.dot(ex, encw[:HD], preferred_element_type=jnp.float32)
      + jnp.dot(oh, encw[HD:], preferred_element_type=jnp.float32)
      + encb_ref[...])


def _node_body(pos_ref, size_ref, ex2_ref, posw_ref, posb_ref, sizew_ref,
               sizeb_ref, nodew_ref, nodeb_ref, node_ref):
  relu = lambda v: jnp.maximum(v, jnp.float32(0.0))
  pos = pos_ref[...]
  size = size_ref[...]
  posw = posw_ref[...]
  sizew = sizew_ref[...]
  pos_h = relu(pos[:, 0:1] * posw[0:1, :] + pos[:, 1:2] * posw[1:2, :]
               + posb_ref[...])
  size_h = relu(size[:, 0:1] * sizew[0:1, :] + size[:, 1:2] * sizew[1:2, :]
                + sizeb_ref[...])
  nodew = nodew_ref[...]
  node_ref[...] = relu(
      jnp.dot(pos_h, nodew[:HD], preferred_element_type=jnp.float32)
      + jnp.dot(size_h, nodew[HD:2 * HD], preferred_element_type=jnp.float32)
      + jnp.dot(ex2_ref[...], nodew[2 * HD:],
                preferred_element_type=jnp.float32)
      + nodeb_ref[...])


def _hp0_body(x_ref, deg_ref, w_ref, hp_ref, dinv_ref):
  deg = jnp.concatenate([deg_ref[0], deg_ref[1]], axis=0)
  dinv = lax.rsqrt(deg + jnp.float32(1.0))
  dinv_ref[...] = dinv
  hp_ref[...] = dinv * jnp.dot(x_ref[...], w_ref[...],
                               preferred_element_type=jnp.float32)


def _hpn_body(h_ref, dinv_ref, w_ref, hp_ref):
  hp_ref[...] = dinv_ref[...] * jnp.dot(h_ref[...], w_ref[...],
                                        preferred_element_type=jnp.float32)


def _relu_body(s_ref, hp_ref, dinv_ref, b_ref, h_ref):
  x = jnp.concatenate([s_ref[0], s_ref[1]], axis=0) + hp_ref[...]
  h_ref[...] = jnp.maximum(dinv_ref[...] * x + b_ref[...], jnp.float32(0.0))


def _segmax_body(h_ref, batch_f_ref, batch_r_ref, g_ref):
  batch_r = batch_r_ref[...]
  g_ref[...] = _segmax(h_ref[...], batch_f_ref[...], batch_r,
                       _start_row(batch_r))


def _head_body(g0_ref, g1_ref, g2_ref, g3_ref, aggw_ref, aggb_ref,
               muw_ref, mub_ref, varw_ref, varb_ref, mu_ref, lv_ref):
  aggw = aggw_ref[...]
  latent = (jnp.dot(g0_ref[...], aggw[:FD],
                    preferred_element_type=jnp.float32)
            + jnp.dot(g1_ref[...], aggw[FD:2 * FD],
                      preferred_element_type=jnp.float32)
            + jnp.dot(g2_ref[...], aggw[2 * FD:3 * FD],
                      preferred_element_type=jnp.float32)
            + jnp.dot(g3_ref[...], aggw[3 * FD:],
                      preferred_element_type=jnp.float32)
            + aggb_ref[...])
  mu_ref[...] = jnp.dot(latent, muw_ref[...],
                        preferred_element_type=jnp.float32) + mub_ref[...]
  lv_ref[...] = jnp.dot(latent, varw_ref[...],
                        preferred_element_type=jnp.float32) + varb_ref[...]


def kernel(pos_features, size_features, exist_features, batch, edge_index,
           pos_W, pos_b, size_W, size_b, exist_table, enc_W, enc_b,
           node_W, node_b, conv_W0, conv_b0, conv_W1, conv_b1,
           conv_W2, conv_b2, agg_W, agg_b, mu_W, mu_b, var_W, var_b):
  f32 = jnp.float32
  row = edge_index[0]
  col = edge_index[1]
  batch_c = batch.reshape(N, 1)
  batch_r = batch.reshape(1, N)
  ex_c = exist_features.reshape(N, 1)
  zeros128 = jnp.zeros((N, FD), f32)
  ones128 = jnp.ones((KCH, FD), f32)
  nf_t = jax.ShapeDtypeStruct((N, FD), f32)
  g_t = jax.ShapeDtypeStruct((B, FD), f32)
  call = lambda body, out: pl.pallas_call(body, out_shape=out)

  deg2 = _sc_degree(col, ones128, zeros128)

  order_r = call(_order_body, jax.ShapeDtypeStruct((1, N), jnp.int32))(
      batch_r)
  ex2 = call(_ex_body, nf_t)(ex_c, exist_table, order_r.reshape(N, 1),
                             enc_W.T, enc_b.reshape(1, FD))
  node = call(_node_body, nf_t)(
      pos_features, size_features, ex2, pos_W.T, pos_b.reshape(1, HD),
      size_W.T, size_b.reshape(1, HD), node_W.T, node_b.reshape(1, FD))
  hp, dinv = call(_hp0_body, [nf_t, nf_t])(node, deg2, conv_W0.T)
  batch_f = jnp.broadcast_to(batch_c, (N, FD))
  g0 = call(_segmax_body, g_t)(node, batch_f, batch_r)

  gs = [g0]
  convs = [(conv_b0, conv_W1), (conv_b1, conv_W2), (conv_b2, None)]
  for b_t, w_next in convs:
    s = _sc_scatter_rows(hp, row, col, zeros128)
    h = call(_relu_body, nf_t)(s, hp, dinv, b_t.reshape(1, FD))
    gs.append(call(_segmax_body, g_t)(h, batch_f, batch_r))
    if w_next is not None:
      hp = call(_hpn_body, nf_t)(h, dinv, w_next.T)

  mu, lv = call(_head_body,
                [jax.ShapeDtypeStruct((B, LD), f32),
                 jax.ShapeDtypeStruct((B, LD), f32)])(
      gs[0], gs[1], gs[2], gs[3], agg_W.T, agg_b.reshape(1, LD),
      mu_W.T, mu_b.reshape(1, LD), var_W.T, var_b.reshape(1, LD))

  return (mu, lv)
